# Initial kernel scaffold; baseline (speedup 1.0000x reference)
import functools
import math

import jax
import jax.numpy as jnp
from jax import lax
from jax.experimental import pallas as pl
from jax.experimental.pallas import tpu as pltpu
from jax.experimental.pallas import tpu_sc as plsc
import numpy as np

N = 1048576
NUM_CAMS = 1000
CPAD = 1024

HALF_PI = math.pi / 2.0
SZ = float(np.sin(np.float32(HALF_PI)))
CZ = float(np.cos(np.float32(HALF_PI)))

NC, NS = 2, 16
NW = NC * NS
RAYS_PER_WORKER = N // NW
B = 4096
CHUNKS = RAYS_PER_WORKER // B
GROUPS = B // 16


def _table_body(angle_ref, dxy_ref, tab_ref):
    a = HALF_PI - angle_ref[0:1, :]
    sa = jnp.sin(a)
    ca = jnp.cos(a)
    vx = -SZ * ca
    vz = -SZ * sa
    n = jnp.sqrt(vx * vx + vz * vz)
    p = vz / n
    q = vx / n
    u = vz * p + vx * q
    tab_ref[0:1, :] = p
    tab_ref[1:2, :] = q
    tab_ref[2:3, :] = u
    tab_ref[3:4, :] = vx
    tab_ref[4:5, :] = vz
    tab_ref[5:6, :] = dxy_ref[0:1, :]
    tab_ref[6:7, :] = dxy_ref[1:2, :]
    tab_ref[7:8, :] = jnp.zeros_like(p)


def _build_table(angle_p, dxy_p):
    return pl.pallas_call(
        _table_body,
        out_shape=jax.ShapeDtypeStruct((8, CPAD), jnp.float32),
    )(angle_p, dxy_p)


def _full(c):
    return jnp.full((16,), c, jnp.int32)


def _sc_body(tab_hbm, ids_hbm, o_hbm, d_hbm, oo_hbm, do_hbm,
             tab_v, ids_v, oo_v, d_v, do_v):
    wid = lax.axis_index("s") * NC + lax.axis_index("c")
    base = wid * RAYS_PER_WORKER
    pltpu.sync_copy(tab_hbm, tab_v)

    def chunk_body(k, carry):
        cb = base + k * B
        pltpu.sync_copy(ids_hbm.at[pl.ds(cb, B)], ids_v)
        pltpu.sync_copy(o_hbm.at[pl.ds(cb, B)], oo_v)
        pltpu.sync_copy(d_hbm.at[pl.ds(cb, B)], d_v)

        def group_body(g, c2):
            s = g * 16
            rows = lax.iota(jnp.int32, 16) + s
            ids = ids_v[pl.ds(s, 16)]
            p = plsc.load_gather(tab_v, [_full(0), ids])
            q = plsc.load_gather(tab_v, [_full(1), ids])
            u = plsc.load_gather(tab_v, [_full(2), ids])
            vx = plsc.load_gather(tab_v, [_full(3), ids])
            vz = plsc.load_gather(tab_v, [_full(4), ids])
            dx = plsc.load_gather(tab_v, [_full(5), ids])
            dy = plsc.load_gather(tab_v, [_full(6), ids])
            d0 = plsc.load_gather(d_v, [rows, _full(0)])
            d1 = plsc.load_gather(d_v, [rows, _full(1)])
            d2 = plsc.load_gather(d_v, [rows, _full(2)])
            cq = CZ * q
            cp = CZ * p
            r0 = p * d0 + cq * d1 + vx * d2
            r1 = u * d1 - CZ * d2
            r2 = cp * d1 + vz * d2 - q * d0
            plsc.store_scatter(do_v, [rows, _full(0)], r0)
            plsc.store_scatter(do_v, [rows, _full(1)], r1)
            plsc.store_scatter(do_v, [rows, _full(2)], r2)
            o0 = plsc.load_gather(oo_v, [rows, _full(0)])
            o1 = plsc.load_gather(oo_v, [rows, _full(1)])
            plsc.store_scatter(oo_v, [rows, _full(0)], o0 + dx)
            plsc.store_scatter(oo_v, [rows, _full(1)], o1 + dy)
            return c2

        lax.fori_loop(0, GROUPS, group_body, 0)
        pltpu.sync_copy(oo_v, oo_hbm.at[pl.ds(cb, B)])
        pltpu.sync_copy(do_v, do_hbm.at[pl.ds(cb, B)])
        return carry

    lax.fori_loop(0, CHUNKS, chunk_body, 0)


_sc_call = functools.partial(
    pl.kernel,
    mesh=plsc.VectorSubcoreMesh(
        core_axis_name="c", subcore_axis_name="s", num_cores=NC, num_subcores=NS
    ),
    out_type=[
        jax.ShapeDtypeStruct((N, 3), jnp.float32),
        jax.ShapeDtypeStruct((N, 3), jnp.float32),
    ],
    scratch_types=[
        pltpu.VMEM((8, CPAD), jnp.float32),
        pltpu.VMEM((B,), jnp.int32),
        pltpu.VMEM((B, 3), jnp.float32),
        pltpu.VMEM((B, 3), jnp.float32),
        pltpu.VMEM((B, 3), jnp.float32),
    ],
)(_sc_body)


@jax.jit
def kernel(rays_o, rays_d, rays_id, angle, dxy):
    ids = rays_id.reshape(-1).astype(jnp.int32)
    angle_p = jnp.zeros((1, CPAD), jnp.float32).at[0, :NUM_CAMS].set(angle)
    dxy_p = jnp.zeros((2, CPAD), jnp.float32).at[:, :NUM_CAMS].set(dxy.T)
    tab = _build_table(angle_p, dxy_p)
    oo, do = _sc_call(tab, ids, rays_o, rays_d)
    return (oo, do)


# speedup vs baseline: 3.7100x; 3.7100x over previous
import functools
import math

import jax
import jax.numpy as jnp
from jax import lax
from jax.experimental import pallas as pl
from jax.experimental.pallas import tpu as pltpu
from jax.experimental.pallas import tpu_sc as plsc
import numpy as np

N = 1048576
NUM_CAMS = 1000
CPAD = 1024

HALF_PI = math.pi / 2.0
SZ = float(np.sin(np.float32(HALF_PI)))
CZ = float(np.cos(np.float32(HALF_PI)))

NC, NS = 2, 16
NW = NC * NS
RAYS_PER_WORKER = N // NW
B = 4096
CHUNKS = RAYS_PER_WORKER // B
GROUPS = B // 16


def _table_body(angle_ref, dxy_ref, tab_ref):
    a = HALF_PI - angle_ref[0:1, :]
    sa = jnp.sin(a)
    ca = jnp.cos(a)
    vx = -SZ * ca
    vz = -SZ * sa
    n = jnp.sqrt(vx * vx + vz * vz)
    p = vz / n
    q = vx / n
    u = vz * p + vx * q
    tab_ref[0:1, :] = p
    tab_ref[1:2, :] = q
    tab_ref[2:3, :] = u
    tab_ref[3:4, :] = vx
    tab_ref[4:5, :] = vz
    tab_ref[5:6, :] = dxy_ref[0:1, :]
    tab_ref[6:7, :] = dxy_ref[1:2, :]
    tab_ref[7:8, :] = jnp.zeros_like(p)


def _build_table(angle_p, dxy_p):
    return pl.pallas_call(
        _table_body,
        out_shape=jax.ShapeDtypeStruct((8, CPAD), jnp.float32),
    )(angle_p, dxy_p)


def _sc_body(tab_hbm, ids_hbm, o_hbm, d_hbm, oo_hbm, do_hbm,
             tab_v, ids_v, oo_v, d_v, do_v):
    wid = lax.axis_index("s") * NC + lax.axis_index("c")
    base = wid * RAYS_PER_WORKER
    pltpu.sync_copy(tab_hbm, tab_v)

    def chunk_body(k, carry):
        cb = base + k * B
        pltpu.sync_copy(ids_hbm.at[pl.ds(cb, B)], ids_v)
        pltpu.sync_copy(o_hbm.at[pl.ds(3 * cb, 3 * B)], oo_v)
        pltpu.sync_copy(d_hbm.at[pl.ds(3 * cb, 3 * B)], d_v)

        def group_body(g, c2):
            s = g * 16
            r0i = lax.iota(jnp.int32, 16) * 3 + (3 * s)
            r1i = r0i + 1
            r2i = r0i + 2
            ids = ids_v[pl.ds(s, 16)]
            p = plsc.load_gather(tab_v, [ids])
            q = plsc.load_gather(tab_v, [ids + (1 * CPAD)])
            u = plsc.load_gather(tab_v, [ids + (2 * CPAD)])
            vx = plsc.load_gather(tab_v, [ids + (3 * CPAD)])
            vz = plsc.load_gather(tab_v, [ids + (4 * CPAD)])
            dx = plsc.load_gather(tab_v, [ids + (5 * CPAD)])
            dy = plsc.load_gather(tab_v, [ids + (6 * CPAD)])
            d0 = plsc.load_gather(d_v, [r0i])
            d1 = plsc.load_gather(d_v, [r1i])
            d2 = plsc.load_gather(d_v, [r2i])
            cq = CZ * q
            cp = CZ * p
            r0 = p * d0 + cq * d1 + vx * d2
            r1 = u * d1 - CZ * d2
            r2 = cp * d1 + vz * d2 - q * d0
            plsc.store_scatter(do_v, [r0i], r0)
            plsc.store_scatter(do_v, [r1i], r1)
            plsc.store_scatter(do_v, [r2i], r2)
            o0 = plsc.load_gather(oo_v, [r0i])
            o1 = plsc.load_gather(oo_v, [r1i])
            plsc.store_scatter(oo_v, [r0i], o0 + dx)
            plsc.store_scatter(oo_v, [r1i], o1 + dy)
            return c2

        lax.fori_loop(0, GROUPS, group_body, 0)
        pltpu.sync_copy(oo_v, oo_hbm.at[pl.ds(3 * cb, 3 * B)])
        pltpu.sync_copy(do_v, do_hbm.at[pl.ds(3 * cb, 3 * B)])
        return carry

    lax.fori_loop(0, CHUNKS, chunk_body, 0)


_sc_call = functools.partial(
    pl.kernel,
    mesh=plsc.VectorSubcoreMesh(
        core_axis_name="c", subcore_axis_name="s", num_cores=NC, num_subcores=NS
    ),
    compiler_params=pltpu.CompilerParams(
        needs_layout_passes=False, use_tc_tiling_on_sc=False
    ),
    out_type=[
        jax.ShapeDtypeStruct((3 * N,), jnp.float32),
        jax.ShapeDtypeStruct((3 * N,), jnp.float32),
    ],
    scratch_types=[
        pltpu.VMEM((8 * CPAD,), jnp.float32),
        pltpu.VMEM((B,), jnp.int32),
        pltpu.VMEM((3 * B,), jnp.float32),
        pltpu.VMEM((3 * B,), jnp.float32),
        pltpu.VMEM((3 * B,), jnp.float32),
    ],
)(_sc_body)


@jax.jit
def kernel(rays_o, rays_d, rays_id, angle, dxy):
    ids = rays_id.reshape(-1).astype(jnp.int32)
    angle_p = jnp.zeros((1, CPAD), jnp.float32).at[0, :NUM_CAMS].set(angle)
    dxy_p = jnp.zeros((2, CPAD), jnp.float32).at[:, :NUM_CAMS].set(dxy.T)
    tab = _build_table(angle_p, dxy_p).reshape(8 * CPAD)
    oo, do = _sc_call(tab, ids, rays_o.reshape(3 * N), rays_d.reshape(3 * N))
    return (oo.reshape(N, 3), do.reshape(N, 3))


# speedup vs baseline: 3.7385x; 1.0077x over previous
import functools
import math

import jax
import jax.numpy as jnp
from jax import lax
from jax.experimental import pallas as pl
from jax.experimental.pallas import tpu as pltpu
from jax.experimental.pallas import tpu_sc as plsc
import numpy as np

N = 1048576
NUM_CAMS = 1000
CPAD = 1024

HALF_PI = math.pi / 2.0
SZ = float(np.sin(np.float32(HALF_PI)))
CZ = float(np.cos(np.float32(HALF_PI)))

NC, NS = 2, 16
NW = NC * NS
RAYS_PER_WORKER = N // NW
B = 4096
CHUNKS = RAYS_PER_WORKER // B
GROUPS = B // 16


def _table_body(angle_ref, dxy_ref, tab_ref):
    a = HALF_PI - angle_ref[0:1, :]
    sa = jnp.sin(a)
    ca = jnp.cos(a)
    vx = -SZ * ca
    vz = -SZ * sa
    n = jnp.sqrt(vx * vx + vz * vz)
    p = vz / n
    q = vx / n
    u = vz * p + vx * q
    tab_ref[0:1, :] = p
    tab_ref[1:2, :] = q
    tab_ref[2:3, :] = u
    tab_ref[3:4, :] = vx
    tab_ref[4:5, :] = vz
    tab_ref[5:6, :] = dxy_ref[0:1, :]
    tab_ref[6:7, :] = dxy_ref[1:2, :]
    tab_ref[7:8, :] = jnp.zeros_like(p)


def _build_table(angle_p, dxy_p):
    return pl.pallas_call(
        _table_body,
        out_shape=jax.ShapeDtypeStruct((8, CPAD), jnp.float32),
    )(angle_p, dxy_p)


def _sc_body(tab_hbm, ids_hbm, o_hbm, d_hbm, oo_hbm, do_hbm,
             tab_v, ids_v, oo_v, d_v, do_v):
    wid = lax.axis_index("s") * NC + lax.axis_index("c")
    base = wid * RAYS_PER_WORKER
    pltpu.sync_copy(tab_hbm, tab_v)

    def chunk_body(k, carry):
        cb = base + k * B
        pltpu.sync_copy(ids_hbm.at[pl.ds(cb, B)], ids_v)
        pltpu.sync_copy(o_hbm.at[pl.ds(3 * cb, 3 * B)], oo_v)
        pltpu.sync_copy(d_hbm.at[pl.ds(3 * cb, 3 * B)], d_v)

        @plsc.parallel_loop(0, B, step=16, unroll=8)
        def group_body(s):
            r0i = lax.iota(jnp.int32, 16) * 3 + (3 * s)
            r1i = r0i + 1
            r2i = r0i + 2
            ids = ids_v[pl.ds(s, 16)]
            p = plsc.load_gather(tab_v, [ids])
            q = plsc.load_gather(tab_v, [ids + (1 * CPAD)])
            u = plsc.load_gather(tab_v, [ids + (2 * CPAD)])
            vx = plsc.load_gather(tab_v, [ids + (3 * CPAD)])
            vz = plsc.load_gather(tab_v, [ids + (4 * CPAD)])
            dx = plsc.load_gather(tab_v, [ids + (5 * CPAD)])
            dy = plsc.load_gather(tab_v, [ids + (6 * CPAD)])
            d0 = plsc.load_gather(d_v, [r0i])
            d1 = plsc.load_gather(d_v, [r1i])
            d2 = plsc.load_gather(d_v, [r2i])
            cq = CZ * q
            cp = CZ * p
            r0 = p * d0 + cq * d1 + vx * d2
            r1 = u * d1 - CZ * d2
            r2 = cp * d1 + vz * d2 - q * d0
            plsc.store_scatter(do_v, [r0i], r0)
            plsc.store_scatter(do_v, [r1i], r1)
            plsc.store_scatter(do_v, [r2i], r2)
            o0 = plsc.load_gather(oo_v, [r0i])
            o1 = plsc.load_gather(oo_v, [r1i])
            plsc.store_scatter(oo_v, [r0i], o0 + dx)
            plsc.store_scatter(oo_v, [r1i], o1 + dy)

        pltpu.sync_copy(oo_v, oo_hbm.at[pl.ds(3 * cb, 3 * B)])
        pltpu.sync_copy(do_v, do_hbm.at[pl.ds(3 * cb, 3 * B)])
        return carry

    lax.fori_loop(0, CHUNKS, chunk_body, 0)


_sc_call = functools.partial(
    pl.kernel,
    mesh=plsc.VectorSubcoreMesh(
        core_axis_name="c", subcore_axis_name="s", num_cores=NC, num_subcores=NS
    ),
    compiler_params=pltpu.CompilerParams(
        needs_layout_passes=False, use_tc_tiling_on_sc=False
    ),
    out_type=[
        jax.ShapeDtypeStruct((3 * N,), jnp.float32),
        jax.ShapeDtypeStruct((3 * N,), jnp.float32),
    ],
    scratch_types=[
        pltpu.VMEM((8 * CPAD,), jnp.float32),
        pltpu.VMEM((B,), jnp.int32),
        pltpu.VMEM((3 * B,), jnp.float32),
        pltpu.VMEM((3 * B,), jnp.float32),
        pltpu.VMEM((3 * B,), jnp.float32),
    ],
)(_sc_body)


@jax.jit
def kernel(rays_o, rays_d, rays_id, angle, dxy):
    ids = rays_id.reshape(-1).astype(jnp.int32)
    angle_p = jnp.zeros((1, CPAD), jnp.float32).at[0, :NUM_CAMS].set(angle)
    dxy_p = jnp.zeros((2, CPAD), jnp.float32).at[:, :NUM_CAMS].set(dxy.T)
    tab = _build_table(angle_p, dxy_p).reshape(8 * CPAD)
    oo, do = _sc_call(tab, ids, rays_o.reshape(3 * N), rays_d.reshape(3 * N))
    return (oo.reshape(N, 3), do.reshape(N, 3))


# speedup vs baseline: 87.8378x; 23.4954x over previous
import functools
import math

import jax
import jax.numpy as jnp
from jax import lax
from jax.experimental import pallas as pl
from jax.experimental.pallas import tpu as pltpu
from jax.experimental.pallas import tpu_sc as plsc
import numpy as np

N = 1048576
NUM_CAMS = 1000
CPAD = 1024

HALF_PI = math.pi / 2.0
SZ = float(np.sin(np.float32(HALF_PI)))
CZ = float(np.cos(np.float32(HALF_PI)))

NC, NS = 2, 16
NW = NC * NS
RAYS_PER_WORKER = N // NW
B = 4096
CHUNKS = RAYS_PER_WORKER // B


def _table_body(angle_ref, dxy_ref, tab_ref):
    a = HALF_PI - angle_ref[0:1, :]
    sa = jnp.sin(a)
    ca = jnp.cos(a)
    vx = -SZ * ca
    vz = -SZ * sa
    n = jnp.sqrt(vx * vx + vz * vz)
    p = vz / n
    q = vx / n
    u = vz * p + vx * q
    tab_ref[0:1, :] = p
    tab_ref[1:2, :] = q
    tab_ref[2:3, :] = u
    tab_ref[3:4, :] = vx
    tab_ref[4:5, :] = vz
    tab_ref[5:6, :] = dxy_ref[0:1, :]
    tab_ref[6:7, :] = dxy_ref[1:2, :]
    tab_ref[7:8, :] = jnp.zeros_like(p)


def _build_table(angle_p, dxy_p):
    return pl.pallas_call(
        _table_body,
        out_shape=jax.ShapeDtypeStruct((8, CPAD), jnp.float32),
    )(angle_p, dxy_p)


def _sc_body(tab_hbm, ids_hbm, o0_hbm, o1_hbm, d0_hbm, d1_hbm, d2_hbm,
             oo0_hbm, oo1_hbm, do0_hbm, do1_hbm, do2_hbm,
             tab_v, ids_v, o0_v, o1_v, d0_v, d1_v, d2_v):
    wid = lax.axis_index("s") * NC + lax.axis_index("c")
    base = wid * RAYS_PER_WORKER
    pltpu.sync_copy(tab_hbm, tab_v)

    def chunk_body(k, carry):
        cb = base + k * B
        sl = pl.ds(cb, B)
        pltpu.sync_copy(ids_hbm.at[sl], ids_v)
        pltpu.sync_copy(o0_hbm.at[sl], o0_v)
        pltpu.sync_copy(o1_hbm.at[sl], o1_v)
        pltpu.sync_copy(d0_hbm.at[sl], d0_v)
        pltpu.sync_copy(d1_hbm.at[sl], d1_v)
        pltpu.sync_copy(d2_hbm.at[sl], d2_v)

        @plsc.parallel_loop(0, B, step=16, unroll=8)
        def group_body(s):
            g = pl.ds(s, 16)
            ids = ids_v[g]
            p = plsc.load_gather(tab_v, [ids])
            q = plsc.load_gather(tab_v, [ids + (1 * CPAD)])
            u = plsc.load_gather(tab_v, [ids + (2 * CPAD)])
            vx = plsc.load_gather(tab_v, [ids + (3 * CPAD)])
            vz = plsc.load_gather(tab_v, [ids + (4 * CPAD)])
            dx = plsc.load_gather(tab_v, [ids + (5 * CPAD)])
            dy = plsc.load_gather(tab_v, [ids + (6 * CPAD)])
            d0 = d0_v[g]
            d1 = d1_v[g]
            d2 = d2_v[g]
            cq = CZ * q
            cp = CZ * p
            r0 = p * d0 + cq * d1 + vx * d2
            r1 = u * d1 - CZ * d2
            r2 = cp * d1 + vz * d2 - q * d0
            d0_v[g] = r0
            d1_v[g] = r1
            d2_v[g] = r2
            o0_v[g] = o0_v[g] + dx
            o1_v[g] = o1_v[g] + dy

        pltpu.sync_copy(o0_v, oo0_hbm.at[sl])
        pltpu.sync_copy(o1_v, oo1_hbm.at[sl])
        pltpu.sync_copy(d0_v, do0_hbm.at[sl])
        pltpu.sync_copy(d1_v, do1_hbm.at[sl])
        pltpu.sync_copy(d2_v, do2_hbm.at[sl])
        return carry

    lax.fori_loop(0, CHUNKS, chunk_body, 0)


_vec = jax.ShapeDtypeStruct((N,), jnp.float32)
_sc_call = functools.partial(
    pl.kernel,
    mesh=plsc.VectorSubcoreMesh(
        core_axis_name="c", subcore_axis_name="s", num_cores=NC, num_subcores=NS
    ),
    compiler_params=pltpu.CompilerParams(
        needs_layout_passes=False, use_tc_tiling_on_sc=False
    ),
    out_type=[_vec, _vec, _vec, _vec, _vec],
    scratch_types=[
        pltpu.VMEM((8 * CPAD,), jnp.float32),
        pltpu.VMEM((B,), jnp.int32),
        pltpu.VMEM((B,), jnp.float32),
        pltpu.VMEM((B,), jnp.float32),
        pltpu.VMEM((B,), jnp.float32),
        pltpu.VMEM((B,), jnp.float32),
        pltpu.VMEM((B,), jnp.float32),
    ],
)(_sc_body)


@jax.jit
def kernel(rays_o, rays_d, rays_id, angle, dxy):
    ids = rays_id.reshape(-1).astype(jnp.int32)
    angle_p = jnp.zeros((1, CPAD), jnp.float32).at[0, :NUM_CAMS].set(angle)
    dxy_p = jnp.zeros((2, CPAD), jnp.float32).at[:, :NUM_CAMS].set(dxy.T)
    tab = _build_table(angle_p, dxy_p).reshape(8 * CPAD)
    o0, o1, o2 = rays_o[:, 0], rays_o[:, 1], rays_o[:, 2]
    d0, d1, d2 = rays_d[:, 0], rays_d[:, 1], rays_d[:, 2]
    oo0, oo1, do0, do1, do2 = _sc_call(tab, ids, o0, o1, d0, d1, d2)
    rays_o_out = jnp.stack([oo0, oo1, o2], axis=-1)
    rays_d_out = jnp.stack([do0, do1, do2], axis=-1)
    return (rays_o_out, rays_d_out)


# speedup vs baseline: 113.9719x; 1.2975x over previous
import functools
import math

import jax
import jax.numpy as jnp
from jax import lax
from jax.experimental import pallas as pl
from jax.experimental.pallas import tpu as pltpu
from jax.experimental.pallas import tpu_sc as plsc
import numpy as np

N = 1048576
NUM_CAMS = 1000
CPAD = 1024

HALF_PI = math.pi / 2.0
SZ = float(np.sin(np.float32(HALF_PI)))
CZ = float(np.cos(np.float32(HALF_PI)))

NC, NS = 2, 16
NW = NC * NS
RAYS_PER_WORKER = N // NW
B = 4096
CHUNKS = RAYS_PER_WORKER // B


def _table_body(angle_ref, dxy_ref, tab_ref):
    a = HALF_PI - angle_ref[0:1, :]
    sa = jnp.sin(a)
    ca = jnp.cos(a)
    vx = -SZ * ca
    vz = -SZ * sa
    n = jnp.sqrt(vx * vx + vz * vz)
    p = vz / n
    q = vx / n
    u = vz * p + vx * q
    tab_ref[0:1, :] = p
    tab_ref[1:2, :] = q
    tab_ref[2:3, :] = u
    tab_ref[3:4, :] = vx
    tab_ref[4:5, :] = vz
    tab_ref[5:6, :] = dxy_ref[0:1, :]
    tab_ref[6:7, :] = dxy_ref[1:2, :]
    tab_ref[7:8, :] = jnp.zeros_like(p)


def _build_table(angle_p, dxy_p):
    return pl.pallas_call(
        _table_body,
        out_shape=jax.ShapeDtypeStruct((8, CPAD), jnp.float32),
    )(angle_p, dxy_p)


def _sc_body(tab_hbm, ids_hbm, o0_hbm, o1_hbm, d0_hbm, d1_hbm, d2_hbm,
             oo0_hbm, oo1_hbm, do0_hbm, do1_hbm, do2_hbm,
             tab_v, ids_v, o0_v, o1_v, d0_v, d1_v, d2_v,
             sem_in, sem_out):
    wid = lax.axis_index("s") * NC + lax.axis_index("c")
    base = wid * RAYS_PER_WORKER
    pltpu.sync_copy(tab_hbm, tab_v)

    ins = (ids_v, o0_v, o1_v, d0_v, d1_v, d2_v)
    in_hbms = (ids_hbm, o0_hbm, o1_hbm, d0_hbm, d1_hbm, d2_hbm)
    out_hbms = (oo0_hbm, oo1_hbm, do0_hbm, do1_hbm, do2_hbm)

    def start_in(k):
        p = k % 2
        sl = pl.ds(base + k * B, B)
        return [
            pltpu.async_copy(h.at[sl], v.at[pl.ds(p * B, B)], sem_in)
            for h, v in zip(in_hbms, ins)
        ]

    def start_out(k):
        p = k % 2
        sl = pl.ds(base + k * B, B)
        outs_v = (o0_v, o1_v, d0_v, d1_v, d2_v)
        return [
            pltpu.async_copy(v.at[pl.ds(p * B, B)], h.at[sl], sem_out)
            for v, h in zip(outs_v, out_hbms)
        ]

    def compute(k):
        p = k % 2
        pb = p * B

        @plsc.parallel_loop(0, B, step=16, unroll=8)
        def group_body(s):
            g = pl.ds(pb + s, 16)
            ids = ids_v[g]
            p_ = plsc.load_gather(tab_v, [ids])
            q = plsc.load_gather(tab_v, [ids + (1 * CPAD)])
            u = plsc.load_gather(tab_v, [ids + (2 * CPAD)])
            vx = plsc.load_gather(tab_v, [ids + (3 * CPAD)])
            vz = plsc.load_gather(tab_v, [ids + (4 * CPAD)])
            dx = plsc.load_gather(tab_v, [ids + (5 * CPAD)])
            dy = plsc.load_gather(tab_v, [ids + (6 * CPAD)])
            d0 = d0_v[g]
            d1 = d1_v[g]
            d2 = d2_v[g]
            cq = CZ * q
            cp = CZ * p_
            r0 = p_ * d0 + cq * d1 + vx * d2
            r1 = u * d1 - CZ * d2
            r2 = cp * d1 + vz * d2 - q * d0
            d0_v[g] = r0
            d1_v[g] = r1
            d2_v[g] = r2
            o0_v[g] = o0_v[g] + dx
            o1_v[g] = o1_v[g] + dy

    pending_out = [None, None]
    in_descs = start_in(0)
    for k in range(CHUNKS):
        if k + 1 < CHUNKS:
            nxt = pending_out[(k + 1) % 2]
            if nxt is not None:
                for d in nxt:
                    d.wait()
                pending_out[(k + 1) % 2] = None
            next_in = start_in(k + 1)
        else:
            next_in = None
        for d in in_descs:
            d.wait()
        compute(k)
        pending_out[k % 2] = start_out(k)
        in_descs = next_in
    for po in pending_out:
        if po is not None:
            for d in po:
                d.wait()


_vec = jax.ShapeDtypeStruct((N,), jnp.float32)
_sc_call = functools.partial(
    pl.kernel,
    mesh=plsc.VectorSubcoreMesh(
        core_axis_name="c", subcore_axis_name="s", num_cores=NC, num_subcores=NS
    ),
    compiler_params=pltpu.CompilerParams(
        needs_layout_passes=False, use_tc_tiling_on_sc=False
    ),
    out_type=[_vec, _vec, _vec, _vec, _vec],
    scratch_types=[
        pltpu.VMEM((8 * CPAD,), jnp.float32),
        pltpu.VMEM((2 * B,), jnp.int32),
        pltpu.VMEM((2 * B,), jnp.float32),
        pltpu.VMEM((2 * B,), jnp.float32),
        pltpu.VMEM((2 * B,), jnp.float32),
        pltpu.VMEM((2 * B,), jnp.float32),
        pltpu.VMEM((2 * B,), jnp.float32),
        pltpu.SemaphoreType.DMA,
        pltpu.SemaphoreType.DMA,
    ],
)(_sc_body)


@jax.jit
def kernel(rays_o, rays_d, rays_id, angle, dxy):
    ids = rays_id.reshape(-1).astype(jnp.int32)
    angle_p = jnp.zeros((1, CPAD), jnp.float32).at[0, :NUM_CAMS].set(angle)
    dxy_p = jnp.zeros((2, CPAD), jnp.float32).at[:, :NUM_CAMS].set(dxy.T)
    tab = _build_table(angle_p, dxy_p).reshape(8 * CPAD)
    o0, o1, o2 = rays_o[:, 0], rays_o[:, 1], rays_o[:, 2]
    d0, d1, d2 = rays_d[:, 0], rays_d[:, 1], rays_d[:, 2]
    oo0, oo1, do0, do1, do2 = _sc_call(tab, ids, o0, o1, d0, d1, d2)
    rays_o_out = jnp.stack([oo0, oo1, o2], axis=-1)
    rays_d_out = jnp.stack([do0, do1, do2], axis=-1)
    return (rays_o_out, rays_d_out)
